# baseline (speedup 1.0000x reference)
import functools

import jax
import jax.numpy as jnp
from jax import lax
from jax.experimental import pallas as pl
from jax.experimental.pallas import tpu as pltpu
from jax.experimental.pallas import tpu_sc as plsc


_ROWS_PER_WORKER = 8


def _sc_gather(pos_table, positions):
    max_len, d_model = pos_table.shape
    n_workers = max_len // _ROWS_PER_WORKER
    info = plsc.get_sparse_core_info()
    mesh = plsc.VectorSubcoreMesh(core_axis_name="c", subcore_axis_name="s")

    @functools.partial(
        pl.kernel,
        mesh=mesh,
        out_type=jax.ShapeDtypeStruct((max_len, d_model), jnp.float32),
        scratch_types=[
            pltpu.VMEM((_ROWS_PER_WORKER,), jnp.int32),
            pltpu.VMEM((_ROWS_PER_WORKER, d_model), jnp.float32),
            pltpu.SemaphoreType.DMA,
        ],
    )
    def gather_kernel(table_hbm, pos_hbm, out_hbm, idx_v, rows_v, sem):
        wid = lax.axis_index("s") * info.num_cores + lax.axis_index("c")

        @pl.when(wid < n_workers)
        def _():
            base = wid * _ROWS_PER_WORKER
            pltpu.sync_copy(pos_hbm.at[pl.ds(base, _ROWS_PER_WORKER)], idx_v)
            pltpu.async_copy(table_hbm.at[idx_v], rows_v, sem).wait()
            pltpu.sync_copy(rows_v, out_hbm.at[pl.ds(base, _ROWS_PER_WORKER)])

    return gather_kernel(pos_table, positions)


_BATCH_BLOCK = 8


def _add_body(x_ref, g_ref, o_ref):
    o_ref[...] = x_ref[...] + g_ref[...][None, :, :]


def _tc_add(x, gathered):
    b, t, c = x.shape
    return pl.pallas_call(
        _add_body,
        grid=(b // _BATCH_BLOCK,),
        in_specs=[
            pl.BlockSpec((_BATCH_BLOCK, t, c), lambda i: (i, 0, 0)),
            pl.BlockSpec((t, c), lambda i: (0, 0)),
        ],
        out_specs=pl.BlockSpec((_BATCH_BLOCK, t, c), lambda i: (i, 0, 0)),
        out_shape=jax.ShapeDtypeStruct((b, t, c), x.dtype),
    )(x, gathered)


def kernel(x, pos_table, positions):
    t = x.shape[1]
    pos = positions.reshape(-1)[:t].astype(jnp.int32)
    gathered = _sc_gather(pos_table, pos)
    return _tc_add(x, gathered)


# speedup vs baseline: 1.0972x; 1.0972x over previous
import functools

import jax
import jax.numpy as jnp
from jax import lax
from jax.experimental import pallas as pl
from jax.experimental.pallas import tpu as pltpu
from jax.experimental.pallas import tpu_sc as plsc


_ROWS_PER_WORKER = 8


def _sc_gather(pos_table, positions):
    max_len, d_model = pos_table.shape
    n_workers = max_len // _ROWS_PER_WORKER
    info = plsc.get_sparse_core_info()
    mesh = plsc.VectorSubcoreMesh(core_axis_name="c", subcore_axis_name="s")

    @functools.partial(
        pl.kernel,
        mesh=mesh,
        out_type=jax.ShapeDtypeStruct((max_len, d_model), jnp.float32),
        scratch_types=[
            pltpu.VMEM((_ROWS_PER_WORKER,), jnp.int32),
            pltpu.VMEM((_ROWS_PER_WORKER, d_model), jnp.float32),
            pltpu.SemaphoreType.DMA,
        ],
    )
    def gather_kernel(table_hbm, pos_hbm, out_hbm, idx_v, rows_v, sem):
        wid = lax.axis_index("s") * info.num_cores + lax.axis_index("c")

        @pl.when(wid < n_workers)
        def _():
            base = wid * _ROWS_PER_WORKER
            pltpu.sync_copy(pos_hbm.at[pl.ds(base, _ROWS_PER_WORKER)], idx_v)
            pltpu.async_copy(table_hbm.at[idx_v], rows_v, sem).wait()
            pltpu.sync_copy(rows_v, out_hbm.at[pl.ds(base, _ROWS_PER_WORKER)])

    return gather_kernel(pos_table, positions)


_BATCH_BLOCK = 8


def _add_body(x_ref, g_ref, o_ref):
    o_ref[...] = x_ref[...] + g_ref[...][None, :, :]


def _tc_add(x, gathered):
    b, t, c = x.shape
    return pl.pallas_call(
        _add_body,
        grid=(b // _BATCH_BLOCK,),
        in_specs=[
            pl.BlockSpec((_BATCH_BLOCK, t, c), lambda i: (i, 0, 0)),
            pl.BlockSpec((t, c), lambda i: (0, 0)),
        ],
        out_specs=pl.BlockSpec((_BATCH_BLOCK, t, c), lambda i: (i, 0, 0)),
        out_shape=jax.ShapeDtypeStruct((b, t, c), x.dtype),
    )(x, gathered)


def kernel(x, pos_table, positions):
    t = x.shape[1]
    pos = positions.reshape(-1)[:t].astype(jnp.int32)
    gathered = jnp.take(pos_table, pos, axis=0)
    return _tc_add(x, gathered)


# speedup vs baseline: 1.2153x; 1.1076x over previous
import functools

import jax
import jax.numpy as jnp
from jax import lax
from jax.experimental import pallas as pl
from jax.experimental.pallas import tpu as pltpu
from jax.experimental.pallas import tpu_sc as plsc


_ROWS_PER_WORKER = 8


def _sc_gather(pos_table, positions):
    max_len, d_model = pos_table.shape
    n_workers = max_len // _ROWS_PER_WORKER
    info = plsc.get_sparse_core_info()
    mesh = plsc.VectorSubcoreMesh(core_axis_name="c", subcore_axis_name="s")

    @functools.partial(
        pl.kernel,
        mesh=mesh,
        out_type=jax.ShapeDtypeStruct((max_len, d_model), jnp.float32),
        scratch_types=[
            pltpu.VMEM((_ROWS_PER_WORKER,), jnp.int32),
            pltpu.VMEM((_ROWS_PER_WORKER, d_model), jnp.float32),
            pltpu.SemaphoreType.DMA,
        ],
    )
    def gather_kernel(table_hbm, pos_hbm, out_hbm, idx_v, rows_v, sem):
        wid = lax.axis_index("s") * info.num_cores + lax.axis_index("c")

        @pl.when(wid < n_workers)
        def _():
            base = wid * _ROWS_PER_WORKER
            pltpu.sync_copy(pos_hbm.at[pl.ds(base, _ROWS_PER_WORKER)], idx_v)
            pltpu.async_copy(table_hbm.at[idx_v], rows_v, sem).wait()
            pltpu.sync_copy(rows_v, out_hbm.at[pl.ds(base, _ROWS_PER_WORKER)])

    return gather_kernel(pos_table, positions)


_BATCH_BLOCK = 32


def _add_body(x_ref, g_ref, o_ref):
    o_ref[...] = x_ref[...] + g_ref[...][None, :, :]


def _tc_add(x, gathered):
    b, t, c = x.shape
    return pl.pallas_call(
        _add_body,
        grid=(b // _BATCH_BLOCK,),
        in_specs=[
            pl.BlockSpec((_BATCH_BLOCK, t, c), lambda i: (i, 0, 0)),
            pl.BlockSpec((t, c), lambda i: (0, 0)),
        ],
        out_specs=pl.BlockSpec((_BATCH_BLOCK, t, c), lambda i: (i, 0, 0)),
        out_shape=jax.ShapeDtypeStruct((b, t, c), x.dtype),
    )(x, gathered)


def kernel(x, pos_table, positions):
    t = x.shape[1]
    pos = positions.reshape(-1)[:t].astype(jnp.int32)
    gathered = jnp.take(pos_table, pos, axis=0)
    return _tc_add(x, gathered)
